# baseline (speedup 1.0000x reference)
import functools

import numpy as np
import jax
import jax.numpy as jnp
from jax import lax
from jax.experimental import pallas as pl
from jax.experimental.pallas import tpu as pltpu
from jax.experimental.pallas import tpu_sc as plsc

_M0 = 0.05
_BY = 0.0625
_LIM = 1.0

_B = 4
_M = 5000
_MP = 5008
_CH = _MP // 16
_N = 1089
_NP = 1104
_NG = _NP // 16
_TCHUNKS = _B * _NG
_NC = 2
_NS = 16
_NW = _NC * _NS
_MAXG = -(-_TCHUNKS // _NW)
_LEVELS = 4
_D2MAX = 8.0


def _make_grid_padded():
    ax = np.arange(-_LIM, _LIM + _BY, _BY, dtype=np.float32)
    g = np.stack(np.meshgrid(ax, ax, indexing="ij"), 0).transpose().reshape(-1, 2)
    assert g.shape[0] == _N
    pad = np.repeat(g[-1:], _NP - _N, axis=0)
    g = np.concatenate([g, pad], 0)
    return g[:, 0].copy(), g[:, 1].copy()


def _body(xs_v, ys_v, w_v, gx_v, gy_v, out_ref,
          xs_t, ys_t, w_t, gx_t, gy_t, d2_t, hw_t, hc_t, ob_t):
    wid = lax.axis_index("s") * _NC + lax.axis_index("c")
    pltpu.sync_copy(xs_v, xs_t)
    pltpu.sync_copy(ys_v, ys_t)
    pltpu.sync_copy(w_v, w_t)
    pltpu.sync_copy(gx_v, gx_t)
    pltpu.sync_copy(gy_v, gy_t)

    lane = lax.iota(jnp.int32, 16)
    zz = jnp.zeros((16,), jnp.float32)
    for l in range(16):
        hw_t[l, pl.ds(0, 16)] = zz
        hc_t[l, pl.ds(0, 16)] = zz

    wbs = []
    for b in range(_B):
        def wsum(k, acc, b=b):
            return acc + w_t[pl.ds(b * _MP + k * 16, 16)]
        acc = lax.fori_loop(0, _CH, wsum, zz)
        wbs.append(jnp.float32(_M0) * jnp.sum(acc))

    def combine(lo, w_base, c_base, level, wbv):
        totw = hw_t[0, pl.ds(0, 16)]
        totc = hc_t[0, pl.ds(0, 16)]
        hw_t[0, pl.ds(0, 16)] = zz
        hc_t[0, pl.ds(0, 16)] = zz
        for l in range(1, 16):
            totw = totw + hw_t[l, pl.ds(0, 16)]
            totc = totc + hc_t[l, pl.ds(0, 16)]
            hw_t[l, pl.ds(0, 16)] = zz
            hc_t[l, pl.ds(0, 16)] = zz
        s = plsc.cumsum(totw)
        sc = plsc.cumsum(totc)
        maskv = s < (wbv - w_base)
        cf = jnp.sum(jnp.where(maskv, jnp.float32(1.0), jnp.float32(0.0)))
        w_prev = jnp.max(jnp.where(maskv, s, jnp.float32(0.0)))
        c_prev = jnp.max(jnp.where(maskv, sc, jnp.float32(0.0)))
        width = jnp.float32(_D2MAX / 16.0 ** level)
        return lo + cf * width, w_base + w_prev, c_base + c_prev

    def group(t, _):
        c = wid + t * _NW

        @pl.when(c < _TCHUNKS)
        def _():
            b = ((c >= _NG).astype(jnp.int32)
                 + (c >= 2 * _NG).astype(jnp.int32)
                 + (c >= 3 * _NG).astype(jnp.int32))
            n0 = (c - b * _NG) * 16
            mbase = b * _MP
            wbv = jnp.where(
                b == 0, wbs[0],
                jnp.where(b == 1, wbs[1], jnp.where(b == 2, wbs[2], wbs[3])))

            gxg = gx_t[pl.ds(n0, 16)]
            gyg = gy_t[pl.ds(n0, 16)]

            def one_point(j, outvec):
                sel = lane == j
                gx = jnp.sum(jnp.where(sel, gxg, jnp.float32(0.0)))
                gy = jnp.sum(jnp.where(sel, gyg, jnp.float32(0.0)))

                inv1 = jnp.float32(16.0 / _D2MAX)

                @plsc.parallel_loop(0, _CH, unroll=4)
                def l1(k):
                    off = pl.ds(mbase + k * 16, 16)
                    xc = xs_t[off]
                    yc = ys_t[off]
                    wc = w_t[off]
                    dx = xc - gx
                    dy = yc - gy
                    d2 = dx * dx + dy * dy
                    d2_t[pl.ds(k * 16, 16)] = d2
                    bins = jnp.minimum((d2 * inv1).astype(jnp.int32), 15)
                    plsc.addupdate_scatter(hw_t, [lane, bins], wc)
                    plsc.addupdate_scatter(hc_t, [lane, bins], wc * d2)
                lo, w_base, c_base = combine(
                    jnp.float32(0.0), jnp.float32(0.0), jnp.float32(0.0), 1, wbv)

                for level in range(2, _LEVELS + 1):
                    inv_w = jnp.float32(16.0 ** level / _D2MAX)

                    @plsc.parallel_loop(0, _CH, unroll=4)
                    def lx(k, inv_w=inv_w, lo=lo):
                        offm = pl.ds(mbase + k * 16, 16)
                        xc = xs_t[offm]
                        yc = ys_t[offm]
                        wc = w_t[offm]
                        dx = xc - gx
                        dy = yc - gy
                        d2 = dx * dx + dy * dy
                        tt = (d2 - lo) * inv_w
                        bins = jnp.minimum(
                            jnp.maximum(tt.astype(jnp.int32), 0), 15)
                        valid = (tt >= 0.0) & (tt < 16.0)
                        wm = jnp.where(valid, wc, jnp.float32(0.0))
                        plsc.addupdate_scatter(hw_t, [lane, bins], wm)
                        plsc.addupdate_scatter(hc_t, [lane, bins], wm * d2)
                    lo, w_base, c_base = combine(lo, w_base, c_base, level, wbv)

                vals = wbv * lo - lo * w_base + c_base
                return jnp.where(lane == j, vals, outvec)

            qv = lax.fori_loop(0, 16, one_point, zz)
            xv = qv * wbv
            iv = plsc.bitcast(xv, jnp.int32)
            y = plsc.bitcast(jnp.int32(0x5F3759DF) - (iv >> 1), jnp.float32)
            for _unused in range(3):
                y = y * (jnp.float32(1.5) - jnp.float32(0.5) * xv * y * y)
            ob_t[:] = qv * y
            pltpu.sync_copy(ob_t, out_ref.at[pl.ds(c * 16, 16)])

        return 0

    lax.fori_loop(0, _MAXG, group, 0)


@functools.cache
def _build():
    mesh = plsc.VectorSubcoreMesh(
        core_axis_name="c", subcore_axis_name="s",
        num_cores=_NC, num_subcores=_NS)
    return pl.kernel(
        _body,
        out_type=jax.ShapeDtypeStruct((_TCHUNKS * 16,), jnp.float32),
        mesh=mesh,
        compiler_params=pltpu.CompilerParams(needs_layout_passes=False),
        scratch_types=[
            pltpu.VMEM((_B * _MP,), jnp.float32),
            pltpu.VMEM((_B * _MP,), jnp.float32),
            pltpu.VMEM((_B * _MP,), jnp.float32),
            pltpu.VMEM((_NP,), jnp.float32),
            pltpu.VMEM((_NP,), jnp.float32),
            pltpu.VMEM((_MP,), jnp.float32),
            pltpu.VMEM((16, 17), jnp.float32),
            pltpu.VMEM((16, 17), jnp.float32),
            pltpu.VMEM((16,), jnp.float32),
        ],
    )


def kernel(inputs, weight):
    gx, gy = _make_grid_padded()
    xs = jnp.pad(inputs[:, :, 0], ((0, 0), (0, _MP - _M))).reshape(-1)
    ys = jnp.pad(inputs[:, :, 1], ((0, 0), (0, _MP - _M))).reshape(-1)
    w = jnp.pad(weight, ((0, 0), (0, _MP - _M))).reshape(-1)
    out = _build()(xs, ys, w, jnp.asarray(gx), jnp.asarray(gy))
    return out.reshape(_B, _NP)[:, :_N]


# speedup vs baseline: 1.0147x; 1.0147x over previous
import functools

import numpy as np
import jax
import jax.numpy as jnp
from jax import lax
from jax.experimental import pallas as pl
from jax.experimental.pallas import tpu as pltpu
from jax.experimental.pallas import tpu_sc as plsc

_M0 = 0.05
_BY = 0.0625
_LIM = 1.0

_B = 4
_M = 5000
_MP = 5008
_CH = _MP // 16
_N = 1089
_NP = 1104
_NG = _NP // 16
_TCHUNKS = _B * _NG
_NC = 2
_NS = 16
_NW = _NC * _NS
_MAXG = -(-_TCHUNKS // _NW)
_LEVELS = 4
_D2MAX = 8.0


def _make_grid_padded():
    ax = np.arange(-_LIM, _LIM + _BY, _BY, dtype=np.float32)
    g = np.stack(np.meshgrid(ax, ax, indexing="ij"), 0).transpose().reshape(-1, 2)
    assert g.shape[0] == _N
    pad = np.repeat(g[-1:], _NP - _N, axis=0)
    g = np.concatenate([g, pad], 0)
    return g[:, 0].copy(), g[:, 1].copy()


def _body(xs_v, ys_v, w_v, gx_v, gy_v, out_ref,
          xs_t, ys_t, w_t, gx_t, gy_t, d2_t, hw_t, hc_t, ob_t):
    wid = lax.axis_index("s") * _NC + lax.axis_index("c")
    pltpu.sync_copy(xs_v, xs_t)
    pltpu.sync_copy(ys_v, ys_t)
    pltpu.sync_copy(w_v, w_t)
    pltpu.sync_copy(gx_v, gx_t)
    pltpu.sync_copy(gy_v, gy_t)

    lane = lax.iota(jnp.int32, 16)
    zz = jnp.zeros((16,), jnp.float32)
    for l in range(16):
        for s in range(4):
            hw_t[l, pl.ds(s * 17, 16)] = zz
            hc_t[l, pl.ds(s * 17, 16)] = zz

    wbs = []
    for b in range(_B):
        def wsum(k, acc, b=b):
            return acc + w_t[pl.ds(b * _MP + k * 16, 16)]
        acc = lax.fori_loop(0, _CH, wsum, zz)
        wbs.append(jnp.float32(_M0) * jnp.sum(acc))

    def combine(lo, w_base, c_base, level, wbv):
        totw = hw_t[0, pl.ds(0, 16)]
        totc = hc_t[0, pl.ds(0, 16)]
        hw_t[0, pl.ds(0, 16)] = zz
        hc_t[0, pl.ds(0, 16)] = zz
        for l in range(16):
            for s in range(4):
                if l == 0 and s == 0:
                    continue
                totw = totw + hw_t[l, pl.ds(s * 17, 16)]
                totc = totc + hc_t[l, pl.ds(s * 17, 16)]
                hw_t[l, pl.ds(s * 17, 16)] = zz
                hc_t[l, pl.ds(s * 17, 16)] = zz
        s = plsc.cumsum(totw)
        sc = plsc.cumsum(totc)
        maskv = s < (wbv - w_base)
        cf = jnp.sum(jnp.where(maskv, jnp.float32(1.0), jnp.float32(0.0)))
        w_prev = jnp.max(jnp.where(maskv, s, jnp.float32(0.0)))
        c_prev = jnp.max(jnp.where(maskv, sc, jnp.float32(0.0)))
        width = jnp.float32(_D2MAX / 16.0 ** level)
        return lo + cf * width, w_base + w_prev, c_base + c_prev

    def group(t, _):
        c = wid + t * _NW

        @pl.when(c < _TCHUNKS)
        def _():
            b = ((c >= _NG).astype(jnp.int32)
                 + (c >= 2 * _NG).astype(jnp.int32)
                 + (c >= 3 * _NG).astype(jnp.int32))
            n0 = (c - b * _NG) * 16
            mbase = b * _MP
            wbv = jnp.where(
                b == 0, wbs[0],
                jnp.where(b == 1, wbs[1], jnp.where(b == 2, wbs[2], wbs[3])))

            gxg = gx_t[pl.ds(n0, 16)]
            gyg = gy_t[pl.ds(n0, 16)]

            def one_point(j, outvec):
                sel = lane == j
                gx = jnp.sum(jnp.where(sel, gxg, jnp.float32(0.0)))
                gy = jnp.sum(jnp.where(sel, gyg, jnp.float32(0.0)))

                inv1 = jnp.float32(16.0 / _D2MAX)

                @plsc.parallel_loop(0, _CH, unroll=4)
                def l1(k):
                    off = pl.ds(mbase + k * 16, 16)
                    xc = xs_t[off]
                    yc = ys_t[off]
                    wc = w_t[off]
                    dx = xc - gx
                    dy = yc - gy
                    d2 = dx * dx + dy * dy
                    d2_t[pl.ds(k * 16, 16)] = d2
                    bins = jnp.minimum((d2 * inv1).astype(jnp.int32), 15)
                    bins = bins + (k & 3) * 17
                    plsc.addupdate_scatter(hw_t, [lane, bins], wc)
                    plsc.addupdate_scatter(hc_t, [lane, bins], wc * d2)
                lo, w_base, c_base = combine(
                    jnp.float32(0.0), jnp.float32(0.0), jnp.float32(0.0), 1, wbv)

                for level in range(2, _LEVELS + 1):
                    inv_w = jnp.float32(16.0 ** level / _D2MAX)

                    @plsc.parallel_loop(0, _CH, unroll=4)
                    def lx(k, inv_w=inv_w, lo=lo):
                        offm = pl.ds(mbase + k * 16, 16)
                        wc = w_t[offm]
                        d2 = d2_t[pl.ds(k * 16, 16)]
                        tt = (d2 - lo) * inv_w
                        bins = jnp.minimum(
                            jnp.maximum(tt.astype(jnp.int32), 0), 15)
                        bins = bins + (k & 3) * 17
                        valid = (tt >= 0.0) & (tt < 16.0)
                        wm = jnp.where(valid, wc, jnp.float32(0.0))
                        plsc.addupdate_scatter(hw_t, [lane, bins], wm)
                        plsc.addupdate_scatter(hc_t, [lane, bins], wm * d2)
                    lo, w_base, c_base = combine(lo, w_base, c_base, level, wbv)

                vals = wbv * lo - lo * w_base + c_base
                return jnp.where(lane == j, vals, outvec)

            qv = lax.fori_loop(0, 16, one_point, zz)
            xv = qv * wbv
            iv = plsc.bitcast(xv, jnp.int32)
            y = plsc.bitcast(jnp.int32(0x5F3759DF) - (iv >> 1), jnp.float32)
            for _unused in range(3):
                y = y * (jnp.float32(1.5) - jnp.float32(0.5) * xv * y * y)
            ob_t[:] = qv * y
            pltpu.sync_copy(ob_t, out_ref.at[pl.ds(c * 16, 16)])

        return 0

    lax.fori_loop(0, _MAXG, group, 0)


@functools.cache
def _build():
    mesh = plsc.VectorSubcoreMesh(
        core_axis_name="c", subcore_axis_name="s",
        num_cores=_NC, num_subcores=_NS)
    return pl.kernel(
        _body,
        out_type=jax.ShapeDtypeStruct((_TCHUNKS * 16,), jnp.float32),
        mesh=mesh,
        compiler_params=pltpu.CompilerParams(needs_layout_passes=False),
        scratch_types=[
            pltpu.VMEM((_B * _MP,), jnp.float32),
            pltpu.VMEM((_B * _MP,), jnp.float32),
            pltpu.VMEM((_B * _MP,), jnp.float32),
            pltpu.VMEM((_NP,), jnp.float32),
            pltpu.VMEM((_NP,), jnp.float32),
            pltpu.VMEM((_MP,), jnp.float32),
            pltpu.VMEM((16, 68), jnp.float32),
            pltpu.VMEM((16, 68), jnp.float32),
            pltpu.VMEM((16,), jnp.float32),
        ],
    )


def kernel(inputs, weight):
    gx, gy = _make_grid_padded()
    xs = jnp.pad(inputs[:, :, 0], ((0, 0), (0, _MP - _M))).reshape(-1)
    ys = jnp.pad(inputs[:, :, 1], ((0, 0), (0, _MP - _M))).reshape(-1)
    w = jnp.pad(weight, ((0, 0), (0, _MP - _M))).reshape(-1)
    out = _build()(xs, ys, w, jnp.asarray(gx), jnp.asarray(gy))
    return out.reshape(_B, _NP)[:, :_N]


# speedup vs baseline: 1.4812x; 1.4597x over previous
import functools

import numpy as np
import jax
import jax.numpy as jnp
from jax import lax
from jax.experimental import pallas as pl
from jax.experimental.pallas import tpu as pltpu
from jax.experimental.pallas import tpu_sc as plsc

_M0 = 0.05
_BY = 0.0625
_LIM = 1.0

_B = 4
_M = 5000
_MP = 5008
_CH = _MP // 16
_N = 1089
_NP = 1104
_NG = _NP // 16
_TCHUNKS = _B * _NG
_NC = 2
_NS = 16
_NW = _NC * _NS
_MAXG = -(-_TCHUNKS // _NW)
_LEVELS = 3
_D2MAX = 8.0


def _make_grid_padded():
    ax = np.arange(-_LIM, _LIM + _BY, _BY, dtype=np.float32)
    g = np.stack(np.meshgrid(ax, ax, indexing="ij"), 0).transpose().reshape(-1, 2)
    assert g.shape[0] == _N
    pad = np.repeat(g[-1:], _NP - _N, axis=0)
    g = np.concatenate([g, pad], 0)
    return g[:, 0].copy(), g[:, 1].copy()


def _body(xs_v, ys_v, w_v, gx_v, gy_v, out_ref,
          xs_t, ys_t, w_t, gx_t, gy_t, d2_t, hw_t, hc_t, ob_t):
    wid = lax.axis_index("s") * _NC + lax.axis_index("c")
    pltpu.sync_copy(xs_v, xs_t)
    pltpu.sync_copy(ys_v, ys_t)
    pltpu.sync_copy(w_v, w_t)
    pltpu.sync_copy(gx_v, gx_t)
    pltpu.sync_copy(gy_v, gy_t)

    lane = lax.iota(jnp.int32, 16)
    zz = jnp.zeros((16,), jnp.float32)
    for l in range(16):
        hw_t[l, pl.ds(0, 16)] = zz
        hc_t[l, pl.ds(0, 16)] = zz

    wbs = []
    for b in range(_B):
        def wsum(k, acc, b=b):
            return acc + w_t[pl.ds(b * _MP + k * 16, 16)]
        acc = lax.fori_loop(0, _CH, wsum, zz)
        wbs.append(jnp.float32(_M0) * jnp.sum(acc))

    def combine(lo, w_base, c_base, level, wbv, want_bin=False):
        totw = hw_t[0, pl.ds(0, 16)]
        totc = hc_t[0, pl.ds(0, 16)]
        hw_t[0, pl.ds(0, 16)] = zz
        hc_t[0, pl.ds(0, 16)] = zz
        for l in range(1, 16):
            totw = totw + hw_t[l, pl.ds(0, 16)]
            totc = totc + hc_t[l, pl.ds(0, 16)]
            hw_t[l, pl.ds(0, 16)] = zz
            hc_t[l, pl.ds(0, 16)] = zz
        s = plsc.cumsum(totw)
        sc = plsc.cumsum(totc)
        maskv = s < (wbv - w_base)
        cf = jnp.sum(jnp.where(maskv, jnp.float32(1.0), jnp.float32(0.0)))
        w_prev = jnp.max(jnp.where(maskv, s, jnp.float32(0.0)))
        c_prev = jnp.max(jnp.where(maskv, sc, jnp.float32(0.0)))
        width = jnp.float32(_D2MAX / 16.0 ** level)
        out = (lo + cf * width, w_base + w_prev, c_base + c_prev)
        if not want_bin:
            return out
        binm = lane == cf.astype(jnp.int32)
        m_w = jnp.sum(jnp.where(binm, totw, jnp.float32(0.0)))
        m_c = jnp.sum(jnp.where(binm, totc, jnp.float32(0.0)))
        return out + (m_w, m_c)

    def group(t, _):
        c = wid + t * _NW

        @pl.when(c < _TCHUNKS)
        def _():
            b = ((c >= _NG).astype(jnp.int32)
                 + (c >= 2 * _NG).astype(jnp.int32)
                 + (c >= 3 * _NG).astype(jnp.int32))
            n0 = (c - b * _NG) * 16
            mbase = b * _MP
            wbv = jnp.where(
                b == 0, wbs[0],
                jnp.where(b == 1, wbs[1], jnp.where(b == 2, wbs[2], wbs[3])))

            gxg = gx_t[pl.ds(n0, 16)]
            gyg = gy_t[pl.ds(n0, 16)]

            def one_point(j, outvec):
                sel = lane == j
                gx = jnp.sum(jnp.where(sel, gxg, jnp.float32(0.0)))
                gy = jnp.sum(jnp.where(sel, gyg, jnp.float32(0.0)))

                inv1 = jnp.float32(16.0 / _D2MAX)

                @plsc.parallel_loop(0, _CH, unroll=4)
                def l1(k):
                    off = pl.ds(mbase + k * 16, 16)
                    xc = xs_t[off]
                    yc = ys_t[off]
                    wc = w_t[off]
                    dx = xc - gx
                    dy = yc - gy
                    d2 = dx * dx + dy * dy
                    d2_t[pl.ds(k * 16, 16)] = d2
                    bins = jnp.minimum((d2 * inv1).astype(jnp.int32), 15)
                    plsc.addupdate_scatter(hw_t, [lane, bins], wc)
                    plsc.addupdate_scatter(hc_t, [lane, bins], wc * d2)
                lo, w_base, c_base = combine(
                    jnp.float32(0.0), jnp.float32(0.0), jnp.float32(0.0), 1, wbv)

                for level in range(2, _LEVELS + 1):
                    inv_w = jnp.float32(16.0 ** level / _D2MAX)

                    @plsc.parallel_loop(0, _CH, unroll=4)
                    def lx(k, inv_w=inv_w, lo=lo):
                        offm = pl.ds(mbase + k * 16, 16)
                        wc = w_t[offm]
                        d2 = d2_t[pl.ds(k * 16, 16)]
                        tt = (d2 - lo) * inv_w
                        bins = jnp.minimum(
                            jnp.maximum(tt.astype(jnp.int32), 0), 15)
                        valid = (tt >= 0.0) & (tt < 16.0)
                        wm = jnp.where(valid, wc, jnp.float32(0.0))
                        plsc.addupdate_scatter(hw_t, [lane, bins], wm)
                        plsc.addupdate_scatter(hc_t, [lane, bins], wm * d2)
                    res = combine(lo, w_base, c_base, level, wbv,
                                  want_bin=(level == _LEVELS))
                    lo, w_base, c_base = res[:3]

                m_w, m_c = res[3], res[4]
                need = jnp.maximum(wbv - w_base, jnp.float32(0.0))
                m_safe = jnp.maximum(m_w, jnp.float32(1e-20))
                rb = lax.bitcast_convert_type(m_safe, jnp.int32)
                rc = lax.bitcast_convert_type(
                    jnp.int32(0x7EF311C3) - rb, jnp.float32)
                for _u in range(3):
                    rc = rc * (jnp.float32(2.0) - m_safe * rc)
                frac = jnp.minimum(need * rc, jnp.float32(1.0))
                width_f = jnp.float32(_D2MAX / 16.0 ** _LEVELS)
                r2 = lo + frac * width_f
                vals = (wbv * r2 - r2 * (w_base + frac * m_w)
                        + (c_base + frac * m_c))
                return jnp.where(lane == j, vals, outvec)

            qv = lax.fori_loop(0, 16, one_point, zz)
            xv = qv * wbv
            iv = plsc.bitcast(xv, jnp.int32)
            y = plsc.bitcast(jnp.int32(0x5F3759DF) - (iv >> 1), jnp.float32)
            for _unused in range(3):
                y = y * (jnp.float32(1.5) - jnp.float32(0.5) * xv * y * y)
            ob_t[:] = qv * y
            pltpu.sync_copy(ob_t, out_ref.at[pl.ds(c * 16, 16)])

        return 0

    lax.fori_loop(0, _MAXG, group, 0)


@functools.cache
def _build():
    mesh = plsc.VectorSubcoreMesh(
        core_axis_name="c", subcore_axis_name="s",
        num_cores=_NC, num_subcores=_NS)
    return pl.kernel(
        _body,
        out_type=jax.ShapeDtypeStruct((_TCHUNKS * 16,), jnp.float32),
        mesh=mesh,
        compiler_params=pltpu.CompilerParams(needs_layout_passes=False),
        scratch_types=[
            pltpu.VMEM((_B * _MP,), jnp.float32),
            pltpu.VMEM((_B * _MP,), jnp.float32),
            pltpu.VMEM((_B * _MP,), jnp.float32),
            pltpu.VMEM((_NP,), jnp.float32),
            pltpu.VMEM((_NP,), jnp.float32),
            pltpu.VMEM((_MP,), jnp.float32),
            pltpu.VMEM((16, 17), jnp.float32),
            pltpu.VMEM((16, 17), jnp.float32),
            pltpu.VMEM((16,), jnp.float32),
        ],
    )


def kernel(inputs, weight):
    gx, gy = _make_grid_padded()
    xs = jnp.pad(inputs[:, :, 0], ((0, 0), (0, _MP - _M))).reshape(-1)
    ys = jnp.pad(inputs[:, :, 1], ((0, 0), (0, _MP - _M))).reshape(-1)
    w = jnp.pad(weight, ((0, 0), (0, _MP - _M))).reshape(-1)
    out = _build()(xs, ys, w, jnp.asarray(gx), jnp.asarray(gy))
    return out.reshape(_B, _NP)[:, :_N]
